# baseline (device time: 94196 ns/iter reference)
import numpy as np

import jax
import jax.numpy as jnp
from jax import lax
from jax.experimental import pallas as pl
from jax.experimental.pallas import tpu as pltpu

N_DEV = 32
W_DEPTH = 4


def _phys_coords(i: int) -> tuple[int, int, int]:
    z, r = divmod(i, 8)
    y, c = divmod(r, 2)
    x = c if y % 2 == 0 else 1 - c
    return x, y, z


def _order_table() -> np.ndarray:
    coords = [_phys_coords(i) for i in range(N_DEV)]
    order = np.zeros((N_DEV, N_DEV), np.int32)
    for m in range(N_DEV):
        def key(p, m=m):
            d = sum(abs(a - b) for a, b in zip(coords[m], coords[p]))
            return (d, (p - m) % N_DEV)
        order[m] = sorted(range(N_DEV), key=key)
    return order


def kernel(x, w_mat):
    m_glob, k_shard = x.shape
    k_glob, n = w_mat.shape
    m_per = m_glob // N_DEV

    x_bf = x.astype(jnp.bfloat16)
    order = jnp.asarray(_order_table())

    def body(x_ref, w_hbm, order_ref, y_ref, xg, w_buf,
             send_sems, recv_sems, w_sems):
        me = lax.axis_index("i")

        sends = []
        for k in range(1, N_DEV):
            tgt = order_ref[me, k]
            rdma = pltpu.make_async_remote_copy(
                src_ref=x_ref.at[pl.ds(tgt * m_per, m_per), :],
                dst_ref=xg.at[me],
                send_sem=send_sems.at[tgt],
                recv_sem=recv_sems.at[me],
                device_id=(tgt,),
                device_id_type=pl.DeviceIdType.MESH,
            )
            rdma.start()
            sends.append(rdma)

        for j in range(W_DEPTH):
            s = order_ref[me, j]
            pltpu.make_async_copy(
                w_hbm.at[pl.ds(s * k_shard, k_shard), :],
                w_buf.at[j],
                w_sems.at[j],
            ).start()

        for j in range(N_DEV):
            s = order_ref[me, j]

            if j == 0:
                x_blk = x_ref[pl.ds(me * m_per, m_per), :]
            else:
                pltpu.make_async_remote_copy(
                    src_ref=x_ref.at[pl.ds(0, m_per), :],
                    dst_ref=xg.at[s],
                    send_sem=send_sems.at[s],
                    recv_sem=recv_sems.at[s],
                    device_id=(me,),
                    device_id_type=pl.DeviceIdType.MESH,
                ).wait_recv()
                x_blk = xg[s]

            pltpu.make_async_copy(
                w_hbm.at[pl.ds(s * k_shard, k_shard), :],
                w_buf.at[j % W_DEPTH],
                w_sems.at[j % W_DEPTH],
            ).wait()

            prod = jnp.dot(
                x_blk.astype(jnp.float32),
                w_buf[j % W_DEPTH],
                preferred_element_type=jnp.float32,
            )
            if j == 0:
                y_ref[:, :] = prod
            else:
                y_ref[:, :] += prod

            if j + W_DEPTH < N_DEV:
                s_next = order_ref[me, j + W_DEPTH]
                pltpu.make_async_copy(
                    w_hbm.at[pl.ds(s_next * k_shard, k_shard), :],
                    w_buf.at[j % W_DEPTH],
                    w_sems.at[j % W_DEPTH],
                ).start()

        for rdma in sends:
            rdma.wait_send()

    return pl.pallas_call(
        body,
        out_shape=jax.ShapeDtypeStruct((m_per, n), jnp.float32),
        in_specs=[
            pl.BlockSpec(memory_space=pltpu.VMEM),
            pl.BlockSpec(memory_space=pl.ANY),
            pl.BlockSpec(memory_space=pltpu.SMEM),
        ],
        out_specs=pl.BlockSpec(memory_space=pltpu.VMEM),
        scratch_shapes=[
            pltpu.VMEM((N_DEV, m_per, k_shard), jnp.bfloat16),
            pltpu.VMEM((W_DEPTH, k_shard, n), w_mat.dtype),
            pltpu.SemaphoreType.DMA((N_DEV,)),
            pltpu.SemaphoreType.DMA((N_DEV,)),
            pltpu.SemaphoreType.DMA((W_DEPTH,)),
        ],
        compiler_params=pltpu.CompilerParams(
            vmem_limit_bytes=64 * 1024 * 1024
        ),
    )(x_bf, w_mat, order)


# device time: 83551 ns/iter; 1.1274x vs baseline; 1.1274x over previous
import numpy as np

import jax
import jax.numpy as jnp
from jax import lax
from jax.experimental import pallas as pl
from jax.experimental.pallas import tpu as pltpu

N_DEV = 32
W_DEPTH = 4
PAY_ROWS = 288


def _phys_coords(i: int) -> tuple[int, int, int]:
    z, r = divmod(i, 8)
    y, c = divmod(r, 2)
    x = c if y % 2 == 0 else 1 - c
    return x, y, z


def _order_table() -> np.ndarray:
    coords = [_phys_coords(i) for i in range(N_DEV)]
    order = np.zeros((N_DEV, N_DEV), np.int32)
    for m in range(N_DEV):
        def key(p, m=m):
            d = sum(abs(a - b) for a, b in zip(coords[m], coords[p]))
            return (d, (p - m) % N_DEV)
        order[m] = sorted(range(N_DEV), key=key)
    return order


def _quantize(x):
    m_glob, k_shard = x.shape
    s = jnp.maximum(jnp.max(jnp.abs(x), axis=0), 1e-30)
    e = jnp.floor(jnp.log2(s))
    m = jnp.ceil(s * jnp.exp2(-e) * 64.0)
    ovf = m > 127.0
    e = jnp.where(ovf, e + 1.0, e)
    m = jnp.where(ovf, 64.0, m)
    s_hat = m * jnp.exp2(e) / 64.0
    q = jnp.clip(jnp.round(x / s_hat * 127.0), -127.0, 127.0)
    q3 = q.astype(jnp.int8).reshape(N_DEV, m_glob // N_DEV, k_shard)
    e_row = jnp.broadcast_to(
        e.astype(jnp.int8)[None, None, :], (N_DEV, 1, k_shard)
    )
    m_row = jnp.broadcast_to(
        m.astype(jnp.int8)[None, None, :], (N_DEV, 1, k_shard)
    )
    pad = jnp.zeros(
        (N_DEV, PAY_ROWS - (m_glob // N_DEV) - 2, k_shard), jnp.int8
    )
    return jnp.concatenate([q3, e_row, m_row, pad], axis=1)


def kernel(x, w_mat):
    m_glob, k_shard = x.shape
    k_glob, n = w_mat.shape
    m_per = m_glob // N_DEV

    payload = _quantize(x)
    order = jnp.asarray(_order_table())

    def body(pay_ref, w_hbm, order_ref, y_ref, xg, w_buf,
             send_sems, recv_sems, w_sems):
        me = lax.axis_index("i")

        sends = []
        for k in range(1, N_DEV):
            tgt = order_ref[me, k]
            rdma = pltpu.make_async_remote_copy(
                src_ref=pay_ref.at[tgt],
                dst_ref=xg.at[me],
                send_sem=send_sems.at[tgt],
                recv_sem=recv_sems.at[me],
                device_id=(tgt,),
                device_id_type=pl.DeviceIdType.MESH,
            )
            rdma.start()
            sends.append(rdma)

        for j in range(W_DEPTH):
            s = order_ref[me, j]
            pltpu.make_async_copy(
                w_hbm.at[pl.ds(s * k_shard, k_shard), :],
                w_buf.at[j],
                w_sems.at[j],
            ).start()

        for j in range(N_DEV):
            s = order_ref[me, j]

            if j == 0:
                q_blk = pay_ref[me, 0:m_per, :]
                e_row = pay_ref[me, m_per:m_per + 1, :]
                m_row = pay_ref[me, m_per + 1:m_per + 2, :]
            else:
                pltpu.make_async_remote_copy(
                    src_ref=pay_ref.at[s],
                    dst_ref=xg.at[s],
                    send_sem=send_sems.at[s],
                    recv_sem=recv_sems.at[s],
                    device_id=(me,),
                    device_id_type=pl.DeviceIdType.MESH,
                ).wait_recv()
                q_blk = xg[s, 0:m_per, :]
                e_row = xg[s, m_per:m_per + 1, :]
                m_row = xg[s, m_per + 1:m_per + 2, :]

            scale = (
                m_row.astype(jnp.float32)
                * jnp.exp2(e_row.astype(jnp.float32))
                * (1.0 / (64.0 * 127.0))
            )
            x_blk = q_blk.astype(jnp.float32) * scale

            pltpu.make_async_copy(
                w_hbm.at[pl.ds(s * k_shard, k_shard), :],
                w_buf.at[j % W_DEPTH],
                w_sems.at[j % W_DEPTH],
            ).wait()

            prod = jnp.dot(
                x_blk, w_buf[j % W_DEPTH],
                preferred_element_type=jnp.float32,
            )
            if j == 0:
                y_ref[:, :] = prod
            else:
                y_ref[:, :] += prod

            if j + W_DEPTH < N_DEV:
                s_next = order_ref[me, j + W_DEPTH]
                pltpu.make_async_copy(
                    w_hbm.at[pl.ds(s_next * k_shard, k_shard), :],
                    w_buf.at[j % W_DEPTH],
                    w_sems.at[j % W_DEPTH],
                ).start()

        for rdma in sends:
            rdma.wait_send()

    return pl.pallas_call(
        body,
        out_shape=jax.ShapeDtypeStruct((m_per, n), jnp.float32),
        in_specs=[
            pl.BlockSpec(memory_space=pltpu.VMEM),
            pl.BlockSpec(memory_space=pl.ANY),
            pl.BlockSpec(memory_space=pltpu.SMEM),
        ],
        out_specs=pl.BlockSpec(memory_space=pltpu.VMEM),
        scratch_shapes=[
            pltpu.VMEM((N_DEV, PAY_ROWS, k_shard), jnp.int8),
            pltpu.VMEM((W_DEPTH, k_shard, n), w_mat.dtype),
            pltpu.SemaphoreType.DMA((N_DEV,)),
            pltpu.SemaphoreType.DMA((N_DEV,)),
            pltpu.SemaphoreType.DMA((W_DEPTH,)),
        ],
        compiler_params=pltpu.CompilerParams(
            vmem_limit_bytes=64 * 1024 * 1024
        ),
    )(payload, w_mat, order)


# device time: 67117 ns/iter; 1.4035x vs baseline; 1.2449x over previous
import numpy as np

import jax
import jax.numpy as jnp
from jax import lax
from jax.experimental import pallas as pl
from jax.experimental.pallas import tpu as pltpu

N_DEV = 32
W_DEPTH = 4
PAY_ROWS = 288


def _phys_coords(i: int) -> tuple[int, int, int]:
    z, r = divmod(i, 8)
    y, c = divmod(r, 2)
    x = c if y % 2 == 0 else 1 - c
    return x, y, z


def _order_table() -> np.ndarray:
    coords = [_phys_coords(i) for i in range(N_DEV)]
    order = np.zeros((N_DEV, N_DEV), np.int32)
    for m in range(N_DEV):
        def key(p, m=m):
            d = sum(abs(a - b) for a, b in zip(coords[m], coords[p]))
            return (d, (p - m) % N_DEV)
        order[m] = sorted(range(N_DEV), key=key)
    return order


def kernel(x, w_mat):
    m_glob, k_shard = x.shape
    k_glob, n = w_mat.shape
    m_per = m_glob // N_DEV

    order = jnp.asarray(_order_table())

    def body(x_ref, w_hbm, order_ref, y_ref, pay, xg, w_buf,
             send_sems, recv_sems, w_sems):
        me = lax.axis_index("i")

        barrier_sem = pltpu.get_barrier_semaphore()
        for off in range(1, N_DEV):
            tgt = lax.rem(me + off, N_DEV)
            pl.semaphore_signal(
                barrier_sem, inc=1,
                device_id=(tgt,), device_id_type=pl.DeviceIdType.MESH,
            )

        for j in range(W_DEPTH):
            s = order_ref[me, j]
            pltpu.make_async_copy(
                w_hbm.at[pl.ds(s * k_shard, k_shard), :],
                w_buf.at[j],
                w_sems.at[j],
            ).start()

        s_col = jnp.maximum(
            jnp.max(jnp.abs(x_ref[:, :]), axis=0), 1e-30
        )
        e_col = jnp.floor(jnp.log2(s_col))
        m_col = jnp.ceil(s_col * jnp.exp2(-e_col) * 64.0)
        ovf = m_col > 127.0
        e_col = jnp.where(ovf, e_col + 1.0, e_col)
        m_col = jnp.where(ovf, 64.0, m_col)
        inv = jnp.exp2(-e_col) * (64.0 * 127.0) / m_col
        e_i8 = e_col.astype(jnp.int8)[None, :]
        m_i8 = m_col.astype(jnp.int8)[None, :]
        for t in range(N_DEV):
            blk = x_ref[t * m_per:(t + 1) * m_per, :]
            q = jnp.clip(jnp.round(blk * inv[None, :]), -127.0, 127.0)
            pay[t, 0:m_per, :] = q.astype(jnp.int8)
            pay[t, m_per:m_per + 1, :] = e_i8
            pay[t, m_per + 1:m_per + 2, :] = m_i8

        s0 = order_ref[me, 0]
        pltpu.make_async_copy(
            w_hbm.at[pl.ds(s0 * k_shard, k_shard), :],
            w_buf.at[0],
            w_sems.at[0],
        ).wait()
        y_ref[:, :] = jnp.dot(
            x_ref[pl.ds(me * m_per, m_per), :],
            w_buf[0],
            preferred_element_type=jnp.float32,
        )
        if W_DEPTH < N_DEV:
            s_next = order_ref[me, W_DEPTH]
            pltpu.make_async_copy(
                w_hbm.at[pl.ds(s_next * k_shard, k_shard), :],
                w_buf.at[0],
                w_sems.at[0],
            ).start()

        pl.semaphore_wait(barrier_sem, N_DEV - 1)

        sends = []
        for k in range(1, N_DEV):
            tgt = order_ref[me, k]
            rdma = pltpu.make_async_remote_copy(
                src_ref=pay.at[tgt],
                dst_ref=xg.at[me],
                send_sem=send_sems.at[tgt],
                recv_sem=recv_sems.at[me],
                device_id=(tgt,),
                device_id_type=pl.DeviceIdType.MESH,
            )
            rdma.start()
            sends.append(rdma)

        for j in range(1, N_DEV):
            s = order_ref[me, j]

            pltpu.make_async_remote_copy(
                src_ref=pay.at[s],
                dst_ref=xg.at[s],
                send_sem=send_sems.at[s],
                recv_sem=recv_sems.at[s],
                device_id=(me,),
                device_id_type=pl.DeviceIdType.MESH,
            ).wait_recv()
            q_blk = xg[s, 0:m_per, :]
            e_row = xg[s, m_per:m_per + 1, :]
            m_row = xg[s, m_per + 1:m_per + 2, :]

            scale = (
                m_row.astype(jnp.float32)
                * jnp.exp2(e_row.astype(jnp.float32))
                * (1.0 / (64.0 * 127.0))
            )
            x_blk = q_blk.astype(jnp.float32) * scale

            pltpu.make_async_copy(
                w_hbm.at[pl.ds(s * k_shard, k_shard), :],
                w_buf.at[j % W_DEPTH],
                w_sems.at[j % W_DEPTH],
            ).wait()

            y_ref[:, :] += jnp.dot(
                x_blk, w_buf[j % W_DEPTH],
                preferred_element_type=jnp.float32,
            )

            if j + W_DEPTH < N_DEV:
                s_next = order_ref[me, j + W_DEPTH]
                pltpu.make_async_copy(
                    w_hbm.at[pl.ds(s_next * k_shard, k_shard), :],
                    w_buf.at[j % W_DEPTH],
                    w_sems.at[j % W_DEPTH],
                ).start()

        for rdma in sends:
            rdma.wait_send()

    return pl.pallas_call(
        body,
        out_shape=jax.ShapeDtypeStruct((m_per, n), jnp.float32),
        in_specs=[
            pl.BlockSpec(memory_space=pltpu.VMEM),
            pl.BlockSpec(memory_space=pl.ANY),
            pl.BlockSpec(memory_space=pltpu.SMEM),
        ],
        out_specs=pl.BlockSpec(memory_space=pltpu.VMEM),
        scratch_shapes=[
            pltpu.VMEM((N_DEV, PAY_ROWS, k_shard), jnp.int8),
            pltpu.VMEM((N_DEV, PAY_ROWS, k_shard), jnp.int8),
            pltpu.VMEM((W_DEPTH, k_shard, n), w_mat.dtype),
            pltpu.SemaphoreType.DMA((N_DEV,)),
            pltpu.SemaphoreType.DMA((N_DEV,)),
            pltpu.SemaphoreType.DMA((W_DEPTH,)),
        ],
        compiler_params=pltpu.CompilerParams(
            vmem_limit_bytes=64 * 1024 * 1024,
            collective_id=0,
        ),
    )(x, w_mat, order)


# device time: 61717 ns/iter; 1.5263x vs baseline; 1.0875x over previous
import numpy as np

import jax
import jax.numpy as jnp
from jax import lax
from jax.experimental import pallas as pl
from jax.experimental.pallas import tpu as pltpu

N_DEV = 32
W_DEPTH = 8
PAY_ROWS = 288


def _phys_coords(i: int) -> tuple[int, int, int]:
    z, r = divmod(i, 8)
    y, c = divmod(r, 2)
    x = c if y % 2 == 0 else 1 - c
    return x, y, z


def _order_table() -> np.ndarray:
    coords = [_phys_coords(i) for i in range(N_DEV)]
    order = np.zeros((N_DEV, N_DEV), np.int32)
    for m in range(N_DEV):
        def key(p, m=m):
            d = sum(abs(a - b) for a, b in zip(coords[m], coords[p]))
            return (d, (p - m) % N_DEV)
        order[m] = sorted(range(N_DEV), key=key)
    return order


def kernel(x, w_mat):
    m_glob, k_shard = x.shape
    k_glob, n = w_mat.shape
    m_per = m_glob // N_DEV

    order = jnp.asarray(_order_table())

    def body(x_ref, w_hbm, order_ref, y_ref, pay, xg, w_buf,
             send_sems, recv_sems, w_sems):
        me = lax.axis_index("i")

        barrier_sem = pltpu.get_barrier_semaphore()
        for off in range(1, N_DEV):
            tgt = lax.rem(me + off, N_DEV)
            pl.semaphore_signal(
                barrier_sem, inc=1,
                device_id=(tgt,), device_id_type=pl.DeviceIdType.MESH,
            )

        for j in range(W_DEPTH):
            s = order_ref[me, j]
            pltpu.make_async_copy(
                w_hbm.at[pl.ds(s * k_shard, k_shard), :],
                w_buf.at[j],
                w_sems.at[j],
            ).start()

        s_col = jnp.maximum(
            jnp.max(jnp.abs(x_ref[:, :]), axis=0), 1e-30
        )
        e_col = jnp.floor(jnp.log2(s_col))
        m_col = jnp.ceil(s_col * jnp.exp2(-e_col) * 64.0)
        ovf = m_col > 127.0
        e_col = jnp.where(ovf, e_col + 1.0, e_col)
        m_col = jnp.where(ovf, 64.0, m_col)
        inv = jnp.exp2(-e_col) * (64.0 * 127.0) / m_col
        e_i8 = e_col.astype(jnp.int8)[None, :]
        m_i8 = m_col.astype(jnp.int8)[None, :]
        for t in range(N_DEV):
            blk = x_ref[t * m_per:(t + 1) * m_per, :]
            q = jnp.clip(jnp.round(blk * inv[None, :]), -127.0, 127.0)
            pay[t, 0:m_per, :] = q.astype(jnp.int8)
            pay[t, m_per:m_per + 1, :] = e_i8
            pay[t, m_per + 1:m_per + 2, :] = m_i8

        s0 = order_ref[me, 0]
        pltpu.make_async_copy(
            w_hbm.at[pl.ds(s0 * k_shard, k_shard), :],
            w_buf.at[0],
            w_sems.at[0],
        ).wait()
        y_ref[:, :] = jnp.dot(
            x_ref[pl.ds(me * m_per, m_per), :],
            w_buf[0],
            preferred_element_type=jnp.float32,
        )
        if W_DEPTH < N_DEV:
            s_next = order_ref[me, W_DEPTH]
            pltpu.make_async_copy(
                w_hbm.at[pl.ds(s_next * k_shard, k_shard), :],
                w_buf.at[0],
                w_sems.at[0],
            ).start()

        pl.semaphore_wait(barrier_sem, N_DEV - 1)

        sends = []
        for k in range(1, N_DEV):
            tgt = order_ref[me, k]
            rdma = pltpu.make_async_remote_copy(
                src_ref=pay.at[tgt],
                dst_ref=xg.at[me],
                send_sem=send_sems.at[tgt],
                recv_sem=recv_sems.at[me],
                device_id=(tgt,),
                device_id_type=pl.DeviceIdType.MESH,
            )
            rdma.start()
            sends.append(rdma)

        for j in range(1, N_DEV):
            s = order_ref[me, j]

            pltpu.make_async_remote_copy(
                src_ref=pay.at[s],
                dst_ref=xg.at[s],
                send_sem=send_sems.at[s],
                recv_sem=recv_sems.at[s],
                device_id=(me,),
                device_id_type=pl.DeviceIdType.MESH,
            ).wait_recv()
            q_blk = xg[s, 0:m_per, :]
            e_row = xg[s, m_per:m_per + 1, :]
            m_row = xg[s, m_per + 1:m_per + 2, :]

            scale = (
                m_row.astype(jnp.float32)
                * jnp.exp2(e_row.astype(jnp.float32))
                * (1.0 / (64.0 * 127.0))
            )
            x_blk = q_blk.astype(jnp.float32) * scale

            pltpu.make_async_copy(
                w_hbm.at[pl.ds(s * k_shard, k_shard), :],
                w_buf.at[j % W_DEPTH],
                w_sems.at[j % W_DEPTH],
            ).wait()

            y_ref[:, :] += jnp.dot(
                x_blk, w_buf[j % W_DEPTH],
                preferred_element_type=jnp.float32,
            )

            if j + W_DEPTH < N_DEV:
                s_next = order_ref[me, j + W_DEPTH]
                pltpu.make_async_copy(
                    w_hbm.at[pl.ds(s_next * k_shard, k_shard), :],
                    w_buf.at[j % W_DEPTH],
                    w_sems.at[j % W_DEPTH],
                ).start()

        for rdma in sends:
            rdma.wait_send()

    return pl.pallas_call(
        body,
        out_shape=jax.ShapeDtypeStruct((m_per, n), jnp.float32),
        in_specs=[
            pl.BlockSpec(memory_space=pltpu.VMEM),
            pl.BlockSpec(memory_space=pl.ANY),
            pl.BlockSpec(memory_space=pltpu.SMEM),
        ],
        out_specs=pl.BlockSpec(memory_space=pltpu.VMEM),
        scratch_shapes=[
            pltpu.VMEM((N_DEV, PAY_ROWS, k_shard), jnp.int8),
            pltpu.VMEM((N_DEV, PAY_ROWS, k_shard), jnp.int8),
            pltpu.VMEM((W_DEPTH, k_shard, n), w_mat.dtype),
            pltpu.SemaphoreType.DMA((N_DEV,)),
            pltpu.SemaphoreType.DMA((N_DEV,)),
            pltpu.SemaphoreType.DMA((W_DEPTH,)),
        ],
        compiler_params=pltpu.CompilerParams(
            vmem_limit_bytes=64 * 1024 * 1024,
            collective_id=0,
        ),
    )(x, w_mat, order)
